# Initial kernel scaffold; baseline (speedup 1.0000x reference)
import jax
import jax.numpy as jnp
from jax.experimental import pallas as pl


def kernel(positions, node_feats, senders, receivers, W_up, W1, W2, w_sh, W_down):
    raise NotImplementedError("write your pallas kernel here")


# speedup vs baseline: 1.1215x; 1.1215x over previous
import functools

import jax
import jax.numpy as jnp
from jax import lax
from jax.experimental import pallas as pl
from jax.experimental.pallas import tpu as pltpu
from jax.experimental.pallas import tpu_sc as plsc

N = 10000
E = 320000
D = 128
CUTOFF = 6.0
NRB = 8
MLP_H = 64
MSG_CH = 2 * D
AVG_NEIGH = 32.0

NC = 2
NS = 16
LANES = 16

_HI = jax.lax.Precision.HIGHEST


def _mm_body(x_ref, w_ref, o_ref):
    o_ref[...] = jax.lax.dot_general(
        x_ref[...], w_ref[...], (((1,), (0,)), ((), ())), precision=_HI)


def _linear_up(node_feats, w_up):
    bn = 2000
    return pl.pallas_call(
        _mm_body,
        grid=(N // bn,),
        in_specs=[
            pl.BlockSpec((bn, D), lambda i: (i, 0)),
            pl.BlockSpec((D, D), lambda i: (0, 0)),
        ],
        out_specs=pl.BlockSpec((bn, D), lambda i: (i, 0)),
        out_shape=jax.ShapeDtypeStruct((N, D), jnp.float32),
    )(node_feats, w_up)


_EB = 80
_PER_W2 = E // (NC * NS)
_NPAD = 10240


def _gather_pos_kernel(px_hbm, py_hbm, pz_hbm, send_hbm, recv_hbm, vec_hbm,
                       px_v, py_v, pz_v, ids_v, idr_v, vbuf_v):
    wid = lax.axis_index("s") * NC + lax.axis_index("c")
    pltpu.sync_copy(px_hbm, px_v.at[pl.ds(0, N)])
    pltpu.sync_copy(py_hbm, py_v.at[pl.ds(0, N)])
    pltpu.sync_copy(pz_hbm, pz_v.at[pl.ds(0, N)])
    nblk = _PER_W2 // _EB

    def body(b, _):
        base = wid * _PER_W2 + b * _EB
        pltpu.sync_copy(send_hbm.at[pl.ds(base, _EB)], ids_v)
        pltpu.sync_copy(recv_hbm.at[pl.ds(base, _EB)], idr_v)

        def grp(g, _):
            s_idx = ids_v[pl.ds(g * LANES, LANES)]
            r_idx = idr_v[pl.ds(g * LANES, LANES)]
            lane = (lax.broadcasted_iota(jnp.int32, (LANES,), 0) * LANES
                    + g * LANES * LANES)
            for c, tab in ((0, px_v), (1, py_v), (2, pz_v)):
                v = (plsc.load_gather(tab, [r_idx])
                     - plsc.load_gather(tab, [s_idx]))
                plsc.store_scatter(vbuf_v, [lane + c], v)
            return 0

        lax.fori_loop(0, _EB // LANES, grp, 0)
        pltpu.sync_copy(vbuf_v, vec_hbm.at[pl.ds(base * LANES, _EB * LANES)])
        return 0

    lax.fori_loop(0, nblk, body, 0)


def _gather_positions(px, py, pz, senders, receivers):
    mesh = plsc.VectorSubcoreMesh(core_axis_name="c", subcore_axis_name="s")
    fn = functools.partial(
        pl.kernel,
        out_type=jax.ShapeDtypeStruct((E * LANES,), jnp.float32),
        mesh=mesh,
        scratch_types=[
            pltpu.VMEM((_NPAD,), jnp.float32),
            pltpu.VMEM((_NPAD,), jnp.float32),
            pltpu.VMEM((_NPAD,), jnp.float32),
            pltpu.VMEM((_EB,), jnp.int32),
            pltpu.VMEM((_EB,), jnp.int32),
            pltpu.VMEM((_EB * LANES,), jnp.float32),
        ],
        compiler_params=pltpu.CompilerParams(needs_layout_passes=False),
    )(_gather_pos_kernel)
    return fn(px, py, pz, senders, receivers)


def _edge_mix_body(vec_ref, w1_ref, w2_ref, wsh_ref, o_ref):
    vec = vec_ref[...][:, :3]
    r2 = jnp.sum(vec * vec, axis=1, keepdims=True) + 1e-9
    r = jnp.sqrt(r2)
    xr = r / CUTOFF

    safe = jnp.where(xr > 1e-6, xr, 1e-6)
    ks = (lax.broadcasted_iota(jnp.int32, (1, NRB), 1) + 1).astype(jnp.float32)
    bes = jnp.sqrt(2.0) * jnp.sin(jnp.pi * ks * xr) / safe

    xc = jnp.clip(xr, 0.0, 0.999)
    x2 = xc * xc
    env = 1.2 * jnp.exp(-2.0 * x2 / (1.0 - x2)) * (xr < 1.0).astype(jnp.float32)
    rb = bes * env

    u = vec / r
    x = u[:, 0:1]
    y = u[:, 1:2]
    z = u[:, 2:3]
    s3 = jnp.sqrt(3.0)
    s15 = jnp.sqrt(15.0)
    s5h = jnp.sqrt(5.0) / 2.0
    sh = jnp.concatenate([
        s3 * x, s3 * y, s3 * z,
        s15 * x * y,
        s15 * y * z,
        s5h * (3.0 * z * z - 1.0),
        s15 * x * z,
        (s15 / 2.0) * (x * x - y * y),
        y * (3.0 * x * x - y * y),
        x * y * z,
        y * (4.0 * z * z - x * x - y * y),
        z * (2.0 * z * z - 3.0 * x * x - 3.0 * y * y),
        x * (4.0 * z * z - x * x - y * y),
        z * (x * x - y * y),
        x * (x * x - 3.0 * y * y),
    ], axis=1)
    gate = jnp.sum(sh * wsh_ref[0:1, :15], axis=1, keepdims=True)

    a = jax.nn.gelu(jax.lax.dot_general(
        rb, w1_ref[...], (((1,), (0,)), ((), ())), precision=_HI))
    mix = jax.lax.dot_general(
        a, w2_ref[...], (((1,), (0,)), ((), ())), precision=_HI)
    o_ref[0] = mix[:, :D]
    o_ref[1] = mix[:, D:] * gate


def _edge_mix(vec, w1, w2, wsh_pad):
    be = 2000
    return pl.pallas_call(
        _edge_mix_body,
        grid=(E // be,),
        in_specs=[
            pl.BlockSpec((be, LANES), lambda i: (i, 0)),
            pl.BlockSpec((NRB, MLP_H), lambda i: (0, 0)),
            pl.BlockSpec((MLP_H, MSG_CH), lambda i: (0, 0)),
            pl.BlockSpec((8, 128), lambda i: (0, 0)),
        ],
        out_specs=pl.BlockSpec((2, be, D), lambda i: (0, i, 0)),
        out_shape=jax.ShapeDtypeStruct((2, E, D), jnp.float32),
    )(vec, w1, w2, wsh_pad)


_PER_S4 = E // NS
_ZROW = 200
_NCHUNK = N // _ZROW


def _scatter_kernel(h_hbm, send_hbm, recv_hbm, mix_hbm, agg_hbm,
                    idx_s, idx_r, msg_v, mix_v, zbuf, sem, agg_sh):
    cid = lax.axis_index("c")
    sid = lax.axis_index("s")

    def zb(k, _):
        rr = k // 8
        jj = (k % 8) * LANES
        zbuf[rr, pl.ds(jj, LANES)] = jnp.zeros((LANES,), jnp.float32)
        return 0
    lax.fori_loop(0, _ZROW * 8, zb, 0)

    def zcp(t, _):
        c = t * NS + sid

        @pl.when(c < _NCHUNK)
        def _():
            pltpu.sync_copy(zbuf, agg_sh.at[pl.ds(c * _ZROW, _ZROW)])
        return 0
    lax.fori_loop(0, (_NCHUNK + NS - 1) // NS, zcp, 0)
    plsc.subcore_barrier()

    nblk = _PER_S4 // _EB

    def body(b, _):
        base = sid * _PER_S4 + b * _EB
        pltpu.sync_copy(send_hbm.at[pl.ds(base, _EB)], idx_s)
        pltpu.sync_copy(recv_hbm.at[pl.ds(base, _EB)], idx_r)
        pltpu.async_copy(h_hbm.at[idx_s], msg_v, sem).wait()
        pltpu.sync_copy(mix_hbm.at[cid, pl.ds(base, _EB), :], mix_v)

        def mul(k, _):
            rr = k // 8
            jj = (k % 8) * LANES
            msg_v[rr, pl.ds(jj, LANES)] = (
                msg_v[rr, pl.ds(jj, LANES)] * mix_v[rr, pl.ds(jj, LANES)])
            return 0
        lax.fori_loop(0, _EB * 8, mul, 0)

        pltpu.sync_copy(msg_v, agg_sh.at[idx_r], add=True)
        return 0

    lax.fori_loop(0, nblk, body, 0)
    plsc.subcore_barrier()

    def ocp(t, _):
        c = t * NS + sid

        @pl.when(c < _NCHUNK)
        def _():
            row0 = c * _ZROW
            pltpu.sync_copy(agg_sh.at[pl.ds(row0, _ZROW)], zbuf)
            pltpu.sync_copy(zbuf, agg_hbm.at[cid, pl.ds(row0, _ZROW), :])
        return 0
    lax.fori_loop(0, (_NCHUNK + NS - 1) // NS, ocp, 0)


def _gather_scatter(h, senders, receivers, mix):
    mesh = plsc.VectorSubcoreMesh(core_axis_name="c", subcore_axis_name="s")
    fn = functools.partial(
        pl.kernel,
        out_type=jax.ShapeDtypeStruct((2, N, D), jnp.float32),
        mesh=mesh,
        scratch_types=[
            pltpu.VMEM((_EB,), jnp.int32),
            pltpu.VMEM((_EB,), jnp.int32),
            pltpu.VMEM((_EB, D), jnp.float32),
            pltpu.VMEM((_EB, D), jnp.float32),
            pltpu.VMEM((_ZROW, D), jnp.float32),
            pltpu.SemaphoreType.DMA,
            pltpu.VMEM_SHARED((N, D), jnp.float32),
        ],
        compiler_params=pltpu.CompilerParams(needs_layout_passes=False),
    )(_scatter_kernel)
    return fn(h, senders, receivers, mix)


def _down_body(agg_ref, wd_ref, o_ref):
    o_ref[...] = (
        jax.lax.dot_general(agg_ref[0], wd_ref[0],
                            (((1,), (0,)), ((), ())), precision=_HI)
        + jax.lax.dot_general(agg_ref[1], wd_ref[1],
                              (((1,), (0,)), ((), ())), precision=_HI))


def _linear_down(agg, wd):
    bn = 2000
    return pl.pallas_call(
        _down_body,
        grid=(N // bn,),
        in_specs=[
            pl.BlockSpec((2, bn, D), lambda i: (0, i, 0)),
            pl.BlockSpec((2, D, D), lambda i: (0, 0, 0)),
        ],
        out_specs=pl.BlockSpec((bn, D), lambda i: (i, 0)),
        out_shape=jax.ShapeDtypeStruct((N, D), jnp.float32),
    )(agg, wd)


def kernel(positions, node_feats, senders, receivers, W_up, W1, W2, w_sh,
           W_down):
    wsh_pad = jnp.zeros((8, 128), jnp.float32).at[0, :15].set(w_sh[:, 0])
    wd = (W_down / jnp.sqrt(AVG_NEIGH)).reshape(2, D, D)

    h = _linear_up(node_feats, W_up)
    vec1d = _gather_positions(positions[:, 0], positions[:, 1],
                              positions[:, 2], senders, receivers)
    mix = _edge_mix(vec1d.reshape(E, LANES), W1, W2, wsh_pad)
    agg = _gather_scatter(h, senders, receivers, mix)
    return _linear_down(agg, wd)
